# baseline (speedup 1.0000x reference)
import jax
import jax.numpy as jnp
from jax.experimental import pallas as pl
from jax.experimental.pallas import tpu as pltpu

_CHUNK = 128


def _embed_kernel(ids_ref, tt_ref, seg_ref, pe_ref, tok_hbm_ref, out_ref,
                  *scratch):
    b = pl.program_id(0)
    S = pe_ref.shape[0]
    n_chunks = S // _CHUNK
    bufs, sems = scratch[:-1], scratch[-1]

    def issue_chunk(c):
        base = c * _CHUNK
        for r in range(_CHUNK):
            pltpu.make_async_copy(
                tok_hbm_ref.at[pl.ds(ids_ref[b, base + r], 1)],
                bufs[c].at[pl.ds(r, 1)],
                sems.at[c]).start()

    def wait_chunk(c):
        pltpu.make_async_copy(
            tok_hbm_ref.at[pl.ds(0, _CHUNK)],
            bufs[c],
            sems.at[c]).wait()

    for c in range(n_chunks):
        issue_chunk(c)
    for c in range(n_chunks):
        wait_chunk(c)
        off = c * _CHUNK
        tok = bufs[c][...]
        tt = tt_ref[0, pl.ds(off, _CHUNK), :]
        seg = jnp.where(tt[:, :, None] == 0, seg_ref[0:1], seg_ref[1:2])
        out_ref[0, pl.ds(off, _CHUNK)] = tok + pe_ref[pl.ds(off, _CHUNK)] + seg


def kernel(input_ids, token_type_ids, tok_table, seg_table, pe):
    B, S = input_ids.shape
    V, D = tok_table.shape
    T = seg_table.shape[0]
    P = D // 128
    n_chunks = S // _CHUNK

    if token_type_ids is None:
        token_type_ids = jnp.zeros_like(input_ids)
    ids = input_ids.astype(jnp.int32)
    tt_3d = token_type_ids.astype(jnp.int32).reshape(B, S, 1)
    tok3 = tok_table.reshape(V, P, 128)
    seg3 = seg_table.reshape(T, P, 128)
    pe3 = pe.reshape(S, P, 128)

    grid_spec = pltpu.PrefetchScalarGridSpec(
        num_scalar_prefetch=1,
        grid=(B,),
        in_specs=[
            pl.BlockSpec((1, S, 1), lambda b, ids_ref: (b, 0, 0)),
            pl.BlockSpec((T, P, 128), lambda b, ids_ref: (0, 0, 0)),
            pl.BlockSpec((S, P, 128), lambda b, ids_ref: (0, 0, 0)),
            pl.BlockSpec(memory_space=pl.ANY),
        ],
        out_specs=pl.BlockSpec((1, S, P, 128), lambda b, ids_ref: (b, 0, 0, 0)),
        scratch_shapes=(
            [pltpu.VMEM((_CHUNK, P, 128), jnp.float32) for _ in range(n_chunks)]
            + [pltpu.SemaphoreType.DMA((n_chunks,))]
        ),
    )
    out = pl.pallas_call(
        _embed_kernel,
        out_shape=jax.ShapeDtypeStruct((B, S, P, 128), jnp.float32),
        grid_spec=grid_spec,
        compiler_params=pltpu.CompilerParams(
            dimension_semantics=("parallel",),
            disable_bounds_checks=True,
        ),
    )(ids, tt_3d, seg3, pe3, tok3)
    return out.reshape(B, S, D)


# speedup vs baseline: 2.9019x; 2.9019x over previous
import jax
import jax.numpy as jnp
from jax.experimental import pallas as pl
from jax.experimental.pallas import tpu as pltpu

_CHUNK = 128


def _embed_kernel(ids_ref, tt_ref, seg_ref, pe_ref, tok_hbm_ref, out_ref,
                  tok_buf, sems):
    b = pl.program_id(0)
    S, D = pe_ref.shape
    n_chunks = S // _CHUNK

    def issue_chunk(c):
        base = c * _CHUNK
        for r in range(_CHUNK):
            pltpu.make_async_copy(
                tok_hbm_ref.at[pl.ds(ids_ref[b, base + r], 1), :],
                tok_buf.at[c, pl.ds(r, 1), :],
                sems.at[c]).start(priority=r % 2)

    def wait_chunk(c):
        pltpu.make_async_copy(
            tok_hbm_ref.at[pl.ds(0, _CHUNK), :],
            tok_buf.at[c],
            sems.at[c]).wait()

    for c in range(n_chunks):
        issue_chunk(c)
    for c in range(n_chunks):
        wait_chunk(c)
        off = c * _CHUNK
        tok = tok_buf[c]
        tt = tt_ref[0, pl.ds(off, _CHUNK), :]
        seg = jnp.where(tt == 0, seg_ref[0:1, :], seg_ref[1:2, :])
        out_ref[0, pl.ds(off, _CHUNK), :] = tok + pe_ref[pl.ds(off, _CHUNK), :] + seg


def kernel(input_ids, token_type_ids, tok_table, seg_table, pe):
    B, S = input_ids.shape
    V, D = tok_table.shape
    T = seg_table.shape[0]
    n_chunks = S // _CHUNK

    if token_type_ids is None:
        token_type_ids = jnp.zeros_like(input_ids)
    ids = input_ids.astype(jnp.int32)
    tt_3d = token_type_ids.astype(jnp.int32).reshape(B, S, 1)

    grid_spec = pltpu.PrefetchScalarGridSpec(
        num_scalar_prefetch=1,
        grid=(B,),
        in_specs=[
            pl.BlockSpec((1, S, 1), lambda b, ids_ref: (b, 0, 0)),
            pl.BlockSpec((T, D), lambda b, ids_ref: (0, 0)),
            pl.BlockSpec((S, D), lambda b, ids_ref: (0, 0)),
            pl.BlockSpec(memory_space=pl.ANY),
        ],
        out_specs=pl.BlockSpec((1, S, D), lambda b, ids_ref: (b, 0, 0)),
        scratch_shapes=[
            pltpu.VMEM((n_chunks, _CHUNK, D), jnp.float32),
            pltpu.SemaphoreType.DMA((n_chunks,)),
        ],
    )
    return pl.pallas_call(
        _embed_kernel,
        out_shape=jax.ShapeDtypeStruct((B, S, D), jnp.float32),
        grid_spec=grid_spec,
        compiler_params=pltpu.CompilerParams(
            dimension_semantics=("parallel",),
            disable_bounds_checks=True,
        ),
    )(ids, tt_3d, seg_table, pe, tok_table)


# speedup vs baseline: 2.9293x; 1.0095x over previous
import jax
import jax.numpy as jnp
from jax.experimental import pallas as pl
from jax.experimental.pallas import tpu as pltpu

_CHUNK = 64


def _embed_kernel(ids_ref, tt_ref, seg_ref, pe_ref, tok_hbm_ref, out_ref,
                  tok_buf, sems):
    b = pl.program_id(0)
    S, D = pe_ref.shape
    n_chunks = S // _CHUNK

    def issue_chunk(c):
        base = c * _CHUNK
        for r in range(_CHUNK):
            pltpu.make_async_copy(
                tok_hbm_ref.at[pl.ds(ids_ref[b, base + r], 1), :],
                tok_buf.at[c, pl.ds(r, 1), :],
                sems.at[c]).start(priority=r % 2)

    def wait_chunk(c):
        pltpu.make_async_copy(
            tok_hbm_ref.at[pl.ds(0, _CHUNK), :],
            tok_buf.at[c],
            sems.at[c]).wait()

    for c in range(n_chunks):
        issue_chunk(c)
    for c in range(n_chunks):
        wait_chunk(c)
        off = c * _CHUNK
        tok = tok_buf[c]
        tt = tt_ref[0, pl.ds(off, _CHUNK), :]
        seg = jnp.where(tt == 0, seg_ref[0:1, :], seg_ref[1:2, :])
        out_ref[0, pl.ds(off, _CHUNK), :] = tok + pe_ref[pl.ds(off, _CHUNK), :] + seg


def kernel(input_ids, token_type_ids, tok_table, seg_table, pe):
    B, S = input_ids.shape
    V, D = tok_table.shape
    T = seg_table.shape[0]
    n_chunks = S // _CHUNK

    if token_type_ids is None:
        token_type_ids = jnp.zeros_like(input_ids)
    ids = input_ids.astype(jnp.int32)
    tt_3d = token_type_ids.astype(jnp.int32).reshape(B, S, 1)

    grid_spec = pltpu.PrefetchScalarGridSpec(
        num_scalar_prefetch=1,
        grid=(B,),
        in_specs=[
            pl.BlockSpec((1, S, 1), lambda b, ids_ref: (b, 0, 0)),
            pl.BlockSpec((T, D), lambda b, ids_ref: (0, 0)),
            pl.BlockSpec((S, D), lambda b, ids_ref: (0, 0)),
            pl.BlockSpec(memory_space=pl.ANY),
        ],
        out_specs=pl.BlockSpec((1, S, D), lambda b, ids_ref: (b, 0, 0)),
        scratch_shapes=[
            pltpu.VMEM((n_chunks, _CHUNK, D), jnp.float32),
            pltpu.SemaphoreType.DMA((n_chunks,)),
        ],
    )
    return pl.pallas_call(
        _embed_kernel,
        out_shape=jax.ShapeDtypeStruct((B, S, D), jnp.float32),
        grid_spec=grid_spec,
        compiler_params=pltpu.CompilerParams(
            dimension_semantics=("parallel",),
            disable_bounds_checks=True,
        ),
    )(ids, tt_3d, seg_table, pe, tok_table)


# speedup vs baseline: 2.9561x; 1.0091x over previous
import jax
import jax.numpy as jnp
from jax.experimental import pallas as pl
from jax.experimental.pallas import tpu as pltpu

_CHUNK = 64


def _embed_kernel(ids_ref, tt_ref, seg_ref, pe_ref, tok_hbm_ref, out_ref,
                  tok_buf, sems):
    core = pl.program_id(0)
    b2 = pl.program_id(1)
    nb2 = pl.num_programs(1)
    S, D = pe_ref.shape
    n_chunks = S // _CHUNK

    row = core * nb2 + b2
    cur = jax.lax.rem(b2, 2)

    def issue_chunk(r_batch, c, buf_set):
        base = c * _CHUNK
        slot = buf_set * n_chunks + c
        for r in range(_CHUNK):
            pltpu.make_async_copy(
                tok_hbm_ref.at[pl.ds(ids_ref[r_batch, base + r], 1), :],
                tok_buf.at[slot, pl.ds(r, 1), :],
                sems.at[slot]).start(priority=r % 2)

    def wait_chunk(slot):
        pltpu.make_async_copy(
            tok_hbm_ref.at[pl.ds(0, _CHUNK), :],
            tok_buf.at[slot, pl.ds(0, _CHUNK), :],
            sems.at[slot]).wait()

    @pl.when(b2 == 0)
    def _():
        for c in range(n_chunks):
            issue_chunk(row, c, cur)

    for c in range(n_chunks):
        @pl.when(b2 + 1 < nb2)
        def _(c=c):
            issue_chunk(row + 1, c, 1 - cur)

        slot = cur * n_chunks + c
        wait_chunk(slot)
        off = c * _CHUNK
        tok = tok_buf[slot]
        tt = tt_ref[0, pl.ds(off, _CHUNK), :]
        seg = jnp.where(tt == 0, seg_ref[0:1, :], seg_ref[1:2, :])
        out_ref[0, pl.ds(off, _CHUNK), :] = tok + pe_ref[pl.ds(off, _CHUNK), :] + seg


def kernel(input_ids, token_type_ids, tok_table, seg_table, pe):
    B, S = input_ids.shape
    V, D = tok_table.shape
    T = seg_table.shape[0]
    n_chunks = S // _CHUNK
    nb2 = B // 2

    if token_type_ids is None:
        token_type_ids = jnp.zeros_like(input_ids)
    ids = input_ids.astype(jnp.int32)
    tt_3d = token_type_ids.astype(jnp.int32).reshape(B, S, 1)

    grid_spec = pltpu.PrefetchScalarGridSpec(
        num_scalar_prefetch=1,
        grid=(2, nb2),
        in_specs=[
            pl.BlockSpec((1, S, 1), lambda i, b2, ids_ref: (i * nb2 + b2, 0, 0)),
            pl.BlockSpec((T, D), lambda i, b2, ids_ref: (0, 0)),
            pl.BlockSpec((S, D), lambda i, b2, ids_ref: (0, 0)),
            pl.BlockSpec(memory_space=pl.ANY),
        ],
        out_specs=pl.BlockSpec((1, S, D), lambda i, b2, ids_ref: (i * nb2 + b2, 0, 0)),
        scratch_shapes=[
            pltpu.VMEM((2 * n_chunks, _CHUNK, D), jnp.float32),
            pltpu.SemaphoreType.DMA((2 * n_chunks,)),
        ],
    )
    return pl.pallas_call(
        _embed_kernel,
        out_shape=jax.ShapeDtypeStruct((B, S, D), jnp.float32),
        grid_spec=grid_spec,
        compiler_params=pltpu.CompilerParams(
            dimension_semantics=("parallel", "arbitrary"),
            disable_bounds_checks=True,
        ),
    )(ids, tt_3d, seg_table, pe, tok_table)
